# Initial kernel scaffold; baseline (speedup 1.0000x reference)
import jax
import jax.numpy as jnp
from jax.experimental import pallas as pl


def kernel(x, edge_index, W1_l, b1, W1_r, W2_l, b2, W2_r, Wo, bo):
    raise NotImplementedError("write your pallas kernel here")


# speedup vs baseline: 9.2333x; 9.2333x over previous
import jax
import jax.numpy as jnp
from jax import lax
from jax.experimental import pallas as pl
from jax.experimental.pallas import tpu as pltpu
from jax.experimental.pallas import tpu_sc as plsc

N = 10000
E = 320000
D_IN = 128
H1 = 64
H2 = 32
D_OUT = 2

NC = 2
NS = 16
NW = NC * NS
EPW = E // NW
C = 80
NSTEPS = EPW // C
NP = 10240
RPT = NP // NS
CNT_W = 16


def _make_segsum(D, with_cnt):
  mesh = plsc.VectorSubcoreMesh(
      core_axis_name="c", subcore_axis_name="s", num_cores=NC, num_subcores=NS)

  out_type = [jax.ShapeDtypeStruct((NC, NP, D), jnp.float32)]
  scratch = [
      pltpu.VMEM((NSTEPS, C), jnp.int32),
      pltpu.VMEM((NSTEPS, C), jnp.int32),
      pltpu.VMEM((C, D), jnp.float32),
      pltpu.VMEM_SHARED((NP, D), jnp.float32),
      pltpu.SemaphoreType.DMA,
  ]
  if with_cnt:
    out_type.append(jax.ShapeDtypeStruct((NC, NP, CNT_W), jnp.float32))
    scratch += [
        pltpu.VMEM((C, CNT_W), jnp.float32),
        pltpu.VMEM_SHARED((NP, CNT_W), jnp.float32),
    ]

  def body(p_hbm, src_hbm, dst_hbm, zrow_hbm, zcnt_hbm, ones_hbm,
           *rest):
    if with_cnt:
      (acc_hbm, cnt_hbm, src_v, dst_v, rows_v, acc_sh, sem,
       ones_v, cnt_sh) = rest
    else:
      (acc_hbm, src_v, dst_v, rows_v, acc_sh, sem) = rest
    c = lax.axis_index("c")
    s = lax.axis_index("s")
    wid = s * NC + c
    r0 = s * RPT

    pltpu.sync_copy(zrow_hbm, acc_sh.at[pl.ds(r0, RPT)])
    if with_cnt:
      pltpu.sync_copy(zcnt_hbm, cnt_sh.at[pl.ds(r0, RPT)])
      pltpu.sync_copy(ones_hbm, ones_v)

    pltpu.sync_copy(src_hbm.at[wid], src_v)
    pltpu.sync_copy(dst_hbm.at[wid], dst_v)
    plsc.subcore_barrier()

    def step(j, carry):
      pltpu.async_copy(p_hbm.at[src_v.at[j]], rows_v, sem).wait()
      pltpu.sync_copy(rows_v, acc_sh.at[dst_v.at[j]], add=True)
      if with_cnt:
        pltpu.sync_copy(ones_v, cnt_sh.at[dst_v.at[j]], add=True)
      return carry

    lax.fori_loop(0, NSTEPS, step, 0)
    plsc.subcore_barrier()

    pltpu.sync_copy(acc_sh.at[pl.ds(r0, RPT)], acc_hbm.at[c, pl.ds(r0, RPT)])
    if with_cnt:
      pltpu.sync_copy(cnt_sh.at[pl.ds(r0, RPT)], cnt_hbm.at[c, pl.ds(r0, RPT)])

  return pl.kernel(
      body, out_type=out_type, mesh=mesh, scratch_types=scratch,
      compiler_params=pltpu.CompilerParams(use_tc_tiling_on_sc=False))


_GRID = 16
_BR = NP // _GRID


def _tc1_body(x_ref, wl_ref, wr_ref, b_ref, p_ref, r_ref):
  xb = x_ref[...]
  p_ref[...] = jnp.dot(xb, wl_ref[...], preferred_element_type=jnp.float32)
  r_ref[...] = (jnp.dot(xb, wr_ref[...], preferred_element_type=jnp.float32)
                + b_ref[...])


def _tc_mid_body(acc_ref, cnt_ref, r_ref, wl_ref, wr_ref, b_ref,
                 p2_ref, r2_ref):
  acc = acc_ref[...]
  cnt = cnt_ref[...]
  tot = jnp.maximum(cnt[0, :, 0:1] + cnt[1, :, 0:1], 1.0)
  h = jnp.maximum((acc[0] + acc[1]) / tot + r_ref[...], 0.0)
  p2_ref[...] = jnp.dot(h, wl_ref[...], preferred_element_type=jnp.float32)
  r2_ref[...] = (jnp.dot(h, wr_ref[...], preferred_element_type=jnp.float32)
                 + b_ref[...])


def _tc3_body(acc_ref, cnt_ref, r_ref, wo_ref, bo_ref, out_ref):
  acc = acc_ref[...]
  cnt = cnt_ref[...]
  tot = jnp.maximum(cnt[0, :, 0:1] + cnt[1, :, 0:1], 1.0)
  h = jnp.maximum((acc[0] + acc[1]) / tot + r_ref[...], 0.0)
  out_ref[...] = (jnp.dot(h, wo_ref[...], preferred_element_type=jnp.float32)
                  + bo_ref[...])


def _row_spec(d):
  return pl.BlockSpec((_BR, d), lambda i: (i, 0))


def _full_spec(shape):
  return pl.BlockSpec(shape, lambda i: tuple(0 for _ in shape))


def _acc_spec(d):
  return pl.BlockSpec((NC, _BR, d), lambda i: (0, i, 0))


def kernel(x, edge_index, W1_l, b1, W1_r, W2_l, b2, W2_r, Wo, bo):
  f32 = jnp.float32
  src_r = edge_index[0].reshape(NW, NSTEPS, C)
  dst_r = edge_index[1].reshape(NW, NSTEPS, C)
  xp = jnp.pad(x, ((0, NP - N), (0, 0)))
  zrow1 = jnp.zeros((RPT, H1), f32)
  zrow2 = jnp.zeros((RPT, H2), f32)
  zcnt = jnp.zeros((RPT, CNT_W), f32)
  ones = jnp.ones((C, CNT_W), f32)

  p1, r1 = pl.pallas_call(
      _tc1_body,
      grid=(_GRID,),
      in_specs=[_row_spec(D_IN), _full_spec((D_IN, H1)), _full_spec((D_IN, H1)),
                _full_spec((1, H1))],
      out_specs=[_row_spec(H1), _row_spec(H1)],
      out_shape=[jax.ShapeDtypeStruct((NP, H1), f32),
                 jax.ShapeDtypeStruct((NP, H1), f32)],
  )(xp, W1_l.T, W1_r.T, b1.reshape(1, H1))

  acc1, cnt = _make_segsum(H1, True)(p1, src_r, dst_r, zrow1, zcnt, ones)

  p2, r2 = pl.pallas_call(
      _tc_mid_body,
      grid=(_GRID,),
      in_specs=[_acc_spec(H1), _acc_spec(CNT_W), _row_spec(H1),
                _full_spec((H1, H2)), _full_spec((H1, H2)), _full_spec((1, H2))],
      out_specs=[_row_spec(H2), _row_spec(H2)],
      out_shape=[jax.ShapeDtypeStruct((NP, H2), f32),
                 jax.ShapeDtypeStruct((NP, H2), f32)],
  )(acc1, cnt, r1, W2_l.T, W2_r.T, b2.reshape(1, H2))

  (acc2,) = _make_segsum(H2, False)(p2, src_r, dst_r, zrow2, zcnt, ones)

  out = pl.pallas_call(
      _tc3_body,
      grid=(_GRID,),
      in_specs=[_acc_spec(H2), _acc_spec(CNT_W), _row_spec(H2),
                _full_spec((H2, D_OUT)), _full_spec((1, D_OUT))],
      out_specs=_row_spec(D_OUT),
      out_shape=jax.ShapeDtypeStruct((NP, D_OUT), f32),
  )(acc2, cnt, r2, Wo.T, bo.reshape(1, D_OUT))
  return out[:N]
